# baseline (speedup 1.0000x reference)
import jax
import jax.numpy as jnp
from jax.experimental import pallas as pl
from jax.experimental.pallas import tpu as pltpu

N = 2048
D = 1024
F = 2048
E = 8
BN = 1024


def _moe_body(x_ref, w1_ref, b1_ref, w2_ref, b2_ref, rt_ref, out_ref, g_scr):
    e = pl.program_id(1)

    @pl.when(e < E)
    def _stage1():
        h = jnp.dot(x_ref[...], w1_ref[0], preferred_element_type=jnp.float32)
        h = h + b1_ref[0]
        h = 0.5 * h * (1.0 + jax.lax.erf(h * 0.7071067811865476))
        g_scr[e % 2] = h.astype(jnp.bfloat16)

    @pl.when(e > 0)
    def _stage2():
        g = g_scr[(e - 1) % 2]
        y = jnp.dot(g, w2_ref[0], preferred_element_type=jnp.float32)
        y = y + b2_ref[0]
        contrib = y * rt_ref[0].reshape(BN, 1)

        @pl.when(e == 1)
        def _init():
            out_ref[...] = contrib

        @pl.when(e != 1)
        def _acc():
            out_ref[...] += contrib


@jax.jit
def kernel(x, routing_tensor, W1, b1, W2, b2):
    b1r = b1.reshape(E, 1, F)
    b2r = b2.reshape(E, 1, D)
    rt = routing_tensor.T.reshape(E, 1, N)

    grid = (N // BN, E + 1)
    ecl = lambda e: jnp.minimum(e, E - 1)
    epr = lambda e: jnp.maximum(e, 1) - 1
    return pl.pallas_call(
        _moe_body,
        grid=grid,
        in_specs=[
            pl.BlockSpec((BN, D), lambda t, e: (t, 0),
                         pipeline_mode=pl.Buffered(buffer_count=1)),
            pl.BlockSpec((1, D, F), lambda t, e: (ecl(e), 0, 0)),
            pl.BlockSpec((1, 1, F), lambda t, e: (ecl(e), 0, 0)),
            pl.BlockSpec((1, F, D), lambda t, e: (epr(e), 0, 0)),
            pl.BlockSpec((1, 1, D), lambda t, e: (epr(e), 0, 0)),
            pl.BlockSpec((1, 1, BN), lambda t, e: (epr(e), 0, t)),
        ],
        out_specs=pl.BlockSpec((BN, D), lambda t, e: (t, 0),
                               pipeline_mode=pl.Buffered(buffer_count=1)),
        out_shape=jax.ShapeDtypeStruct((N, D), jnp.float32),
        scratch_shapes=[pltpu.VMEM((2, BN, F), jnp.bfloat16)],
        compiler_params=pltpu.CompilerParams(
            dimension_semantics=("parallel", "arbitrary"),
        ),
    )(x, W1, b1r, W2, b2r, rt)


# speedup vs baseline: 1.0365x; 1.0365x over previous
import jax
import jax.numpy as jnp
from jax.experimental import pallas as pl
from jax.experimental.pallas import tpu as pltpu

N = 2048
D = 1024
F = 2048
E = 8
FC = 2
FB = F // FC


def _moe_body(x_ref, w1_ref, b1_ref, w2_ref, b2_ref, rt_ref, out_ref):
    e = pl.program_id(0)
    fc = pl.program_id(1)
    h = jnp.dot(x_ref[...], w1_ref[0], preferred_element_type=jnp.float32)
    h = h + b1_ref[0]
    h = 0.5 * h * (1.0 + jax.lax.erf(h * 0.7071067811865476))
    y = jnp.dot(h, w2_ref[0], preferred_element_type=jnp.float32)
    y = y + b2_ref[0] * jnp.where(fc == 0, 1.0, 0.0)
    contrib = y * rt_ref[0].reshape(N, 1)

    @pl.when((e == 0) & (fc == 0))
    def _init():
        out_ref[...] = contrib

    @pl.when((e > 0) | (fc > 0))
    def _acc():
        out_ref[...] += contrib


@jax.jit
def kernel(x, routing_tensor, W1, b1, W2, b2):
    b1r = b1.reshape(E, 1, F)
    b2r = b2.reshape(E, 1, D)
    rt = routing_tensor.T.reshape(E, 1, N)

    return pl.pallas_call(
        _moe_body,
        grid=(E, FC),
        in_specs=[
            pl.BlockSpec((N, D), lambda e, fc: (0, 0),
                         pipeline_mode=pl.Buffered(buffer_count=1)),
            pl.BlockSpec((1, D, FB), lambda e, fc: (e, 0, fc)),
            pl.BlockSpec((1, 1, FB), lambda e, fc: (e, 0, fc)),
            pl.BlockSpec((1, FB, D), lambda e, fc: (e, fc, 0)),
            pl.BlockSpec((1, 1, D), lambda e, fc: (e, 0, 0)),
            pl.BlockSpec((1, 1, N), lambda e, fc: (e, 0, 0)),
        ],
        out_specs=pl.BlockSpec((N, D), lambda e, fc: (0, 0),
                               pipeline_mode=pl.Buffered(buffer_count=1)),
        out_shape=jax.ShapeDtypeStruct((N, D), jnp.float32),
        compiler_params=pltpu.CompilerParams(
            dimension_semantics=("arbitrary", "arbitrary"),
        ),
    )(x, W1, b1r, W2, b2r, rt)


# speedup vs baseline: 1.1529x; 1.1124x over previous
import jax
import jax.numpy as jnp
from jax.experimental import pallas as pl
from jax.experimental.pallas import tpu as pltpu

N = 2048
D = 1024
F = 2048
E = 8
BN = 1024


def _moe_body(x_ref, w1_ref, b1_ref, w2_ref, b2_ref, rt_ref, out_ref):
    e = pl.program_id(1)
    h = jnp.dot(x_ref[...], w1_ref[0], preferred_element_type=jnp.float32)
    h = h + b1_ref[0]
    h = 0.5 * h * (1.0 + jax.lax.erf(h * 0.7071067811865476))
    y = jnp.dot(h, w2_ref[0], preferred_element_type=jnp.float32)
    y = y + b2_ref[0]
    scale = rt_ref[0].reshape(BN, 1)
    contrib = y * scale

    @pl.when(e == 0)
    def _init():
        out_ref[...] = contrib

    @pl.when(e != 0)
    def _acc():
        out_ref[...] += contrib


@jax.jit
def kernel(x, routing_tensor, W1, b1, W2, b2):
    b1r = b1.reshape(E, 1, F)
    b2r = b2.reshape(E, 1, D)
    rt = routing_tensor.T.reshape(E, 1, N)

    grid = (N // BN, E)
    return pl.pallas_call(
        _moe_body,
        grid=grid,
        in_specs=[
            pl.BlockSpec((BN, D), lambda t, e: (t, 0)),
            pl.BlockSpec((1, D, F), lambda t, e: (e, 0, 0)),
            pl.BlockSpec((1, 1, F), lambda t, e: (e, 0, 0)),
            pl.BlockSpec((1, F, D), lambda t, e: (e, 0, 0)),
            pl.BlockSpec((1, 1, D), lambda t, e: (e, 0, 0)),
            pl.BlockSpec((1, 1, BN), lambda t, e: (e, 0, t)),
        ],
        out_specs=pl.BlockSpec((BN, D), lambda t, e: (t, 0)),
        out_shape=jax.ShapeDtypeStruct((N, D), jnp.float32),
        compiler_params=pltpu.CompilerParams(
            dimension_semantics=("parallel", "arbitrary"),
        ),
    )(x, W1, b1r, W2, b2r, rt)
